# Initial kernel scaffold; baseline (speedup 1.0000x reference)
import jax
import jax.numpy as jnp
from jax.experimental import pallas as pl


def kernel(in_weights, out_weights, activation_values, utility, replacement_accumulator, age):
    raise NotImplementedError("write your pallas kernel here")


# speedup vs baseline: 2.2238x; 2.2238x over previous
import functools

import numpy as np
import jax
import jax.numpy as jnp
from jax import lax
from jax.experimental import pallas as pl
from jax.experimental.pallas import tpu as pltpu

REPLACE_RATE = 1e-4
DECAY_RATE = 0.99
MATURITY_THRESHOLD = 100

_ROT = ((13, 15, 26, 6), (17, 29, 16, 24))


def _np_threefry2x32(k0, k1, x0, x1):
    x0 = x0.astype(np.uint32).copy()
    x1 = x1.astype(np.uint32).copy()
    ks = [np.uint32(k0), np.uint32(k1),
          np.uint32(np.uint32(k0) ^ np.uint32(k1) ^ np.uint32(0x1BD11BDA))]
    x0 = (x0 + ks[0]).astype(np.uint32)
    x1 = (x1 + ks[1]).astype(np.uint32)
    for i in range(5):
        for r in _ROT[i % 2]:
            x0 = (x0 + x1).astype(np.uint32)
            x1 = ((x1 << np.uint32(r)) | (x1 >> np.uint32(32 - r))).astype(np.uint32)
            x1 = (x1 ^ x0).astype(np.uint32)
        x0 = (x0 + ks[(i + 1) % 3]).astype(np.uint32)
        x1 = (x1 + ks[(i + 2) % 3] + np.uint32(i + 1)).astype(np.uint32)
    return x0, x1


def _in_key():
    b1, b2 = _np_threefry2x32(np.uint32(0), np.uint32(42),
                              np.array([0, 0], np.uint32), np.array([0, 1], np.uint32))
    return int(b1[0]), int(b2[0])


def _i32(v):
    return np.int32(np.uint32(v & 0xFFFFFFFF))


def _tf_bits(idx, k0, k1):
    ks = [_i32(k0), _i32(k1), _i32(k0 ^ k1 ^ 0x1BD11BDA)]
    x0 = jnp.full(idx.shape, ks[0], jnp.int32)
    x1 = idx + ks[1]
    for i in range(5):
        for r in _ROT[i % 2]:
            x0 = x0 + x1
            x1 = lax.shift_left(x1, np.int32(r)) | lax.shift_right_logical(x1, np.int32(32 - r))
            x1 = lax.bitwise_xor(x1, x0)
        x0 = x0 + ks[(i + 1) % 3]
        x1 = x1 + ks[(i + 2) % 3] + np.int32(i + 1)
    return lax.bitwise_xor(x0, x1)


_POS_INF_BITS = np.int32(0x7F800000)


def _kth_smallest_bits(bits, k):
    def body(_, lohi):
        lo, hi = lohi
        mid = lo + lax.div(hi - lo, jnp.int32(2))
        cnt = jnp.sum((bits <= mid).astype(jnp.int32))
        ge = cnt >= k
        return (jnp.where(ge, lo, mid + 1), jnp.where(ge, mid, hi))
    _, hi = lax.fori_loop(0, 31, body, (jnp.int32(0), _POS_INF_BITS))
    return hi


def _stats_kernel(batch, n_features, ow_ref, act_ref, util_ref, age_ref, racc_ref,
                  ru_ref, racc_out_ref, rage_ref, mask_ref, ws_ref, im_ref):
    i = pl.program_id(0)
    s = jnp.sum(jnp.abs(ow_ref[...]), axis=0, keepdims=True)
    a = jnp.sum(jnp.abs(act_ref[...]), axis=0, keepdims=True)

    @pl.when(i == 0)
    def _():
        ws_ref[...] = s
        im_ref[...] = a

    @pl.when(i > 0)
    def _():
        ws_ref[...] = ws_ref[...] + s
        im_ref[...] = im_ref[...] + a

    @pl.when(i == pl.num_programs(0) - 1)
    def _():
        im = im_ref[...] * np.float32(1.0 / batch)
        step_u = im * ws_ref[...]
        new_u = np.float32(1.0 - DECAY_RATE) * step_u + np.float32(DECAY_RATE) * util_ref[...]
        new_age = age_ref[...] + 1
        elig = new_age > MATURITY_THRESHOLD
        n_elig = jnp.sum(elig.astype(jnp.int32))
        racc1 = racc_ref[...][0, 0] + np.float32(REPLACE_RATE * n_features)
        n_av = racc1.astype(jnp.int32)
        k = jnp.minimum(n_av, n_elig)

        ubits = lax.bitcast_convert_type(new_u, jnp.int32)
        fbits = jnp.where(elig, ubits, _POS_INF_BITS)
        tbits = _kth_smallest_bits(fbits, k)
        m_lo = _kth_smallest_bits(ubits, jnp.int32(n_features // 2))
        m_hi = _kth_smallest_bits(ubits, jnp.int32(n_features // 2 + 1))
        med = (lax.bitcast_convert_type(m_lo, jnp.float32)
               + lax.bitcast_convert_type(m_hi, jnp.float32)) * np.float32(0.5)

        pm = jnp.logical_and(jnp.logical_and(n_av > 0, elig), fbits <= tbits)
        ru_ref[...] = jnp.where(pm, med, new_u)
        rage_ref[...] = jnp.where(pm, 0, new_age)
        mask_ref[...] = pm.astype(jnp.int32)
        racc2 = racc1 - jnp.where(n_av > 0, k, 0).astype(jnp.float32)
        racc_out_ref[...] = jnp.full((1, 1), racc2, jnp.float32)


def _rewrite_kernel(block_rows, n_cols, k0, k1, limit,
                    iw_ref, ow_ref, rm_ref, cm_ref, iw_out_ref, ow_out_ref):
    i = pl.program_id(0)
    ow_out_ref[...] = jnp.where(cm_ref[...] != 0, np.float32(0.0), ow_ref[...])

    rm = rm_ref[...]
    any_m = jnp.sum(rm) > 0

    @pl.when(jnp.logical_not(any_m))
    def _():
        iw_out_ref[...] = iw_ref[...]

    @pl.when(any_m)
    def _():
        base = i * np.int32(block_rows * n_cols)
        idx = (base
               + lax.broadcasted_iota(jnp.int32, (block_rows, n_cols), 0) * np.int32(n_cols)
               + lax.broadcasted_iota(jnp.int32, (block_rows, n_cols), 1))
        bits = _tf_bits(idx, k0, k1)
        fb = lax.shift_right_logical(bits, np.int32(9)) | np.int32(0x3F800000)
        f = lax.bitcast_convert_type(fb, jnp.float32) - np.float32(1.0)
        rngv = jnp.maximum(np.float32(-limit),
                           f * np.float32(2.0 * limit) + np.float32(-limit))
        iw_out_ref[...] = jnp.where(rm != 0, rngv, iw_ref[...])


def kernel(in_weights, out_weights, activation_values, utility, replacement_accumulator, age):
    n_features = out_weights.shape[1]
    out_features = out_weights.shape[0]
    in_features = in_weights.shape[1]
    batch = activation_values.shape[0]

    g1 = 8
    ow_rows = out_features // g1
    act_rows = batch // g1

    stats = pl.pallas_call(
        functools.partial(_stats_kernel, batch, n_features),
        grid=(g1,),
        in_specs=[
            pl.BlockSpec((ow_rows, n_features), lambda i: (i, 0)),
            pl.BlockSpec((act_rows, n_features), lambda i: (i, 0)),
            pl.BlockSpec((1, n_features), lambda i: (0, 0)),
            pl.BlockSpec((1, n_features), lambda i: (0, 0)),
            pl.BlockSpec((1, 1), lambda i: (0, 0)),
        ],
        out_specs=[
            pl.BlockSpec((1, n_features), lambda i: (0, 0)),
            pl.BlockSpec((1, 1), lambda i: (0, 0)),
            pl.BlockSpec((1, n_features), lambda i: (0, 0)),
            pl.BlockSpec((1, n_features), lambda i: (0, 0)),
        ],
        out_shape=[
            jax.ShapeDtypeStruct((1, n_features), jnp.float32),
            jax.ShapeDtypeStruct((1, 1), jnp.float32),
            jax.ShapeDtypeStruct((1, n_features), jnp.int32),
            jax.ShapeDtypeStruct((1, n_features), jnp.int32),
        ],
        scratch_shapes=[
            pltpu.VMEM((1, n_features), jnp.float32),
            pltpu.VMEM((1, n_features), jnp.float32),
        ],
    )(
        out_weights,
        activation_values,
        utility.reshape(1, n_features),
        age.reshape(1, n_features),
        replacement_accumulator.reshape(1, 1),
    )
    ru, racc_out, rage, mask = stats

    row_mask = mask.reshape(n_features, 1)
    k0, k1 = _in_key()
    limit = float(np.sqrt(np.float32(3.0) / np.float32(in_features)))

    r2 = 64
    g2 = out_features // r2
    iw_new, ow_new = pl.pallas_call(
        functools.partial(_rewrite_kernel, r2, in_features, k0, k1, limit),
        grid=(g2,),
        in_specs=[
            pl.BlockSpec((r2, in_features), lambda i: (i, 0)),
            pl.BlockSpec((r2, n_features), lambda i: (i, 0)),
            pl.BlockSpec((r2, 1), lambda i: (i, 0)),
            pl.BlockSpec((1, n_features), lambda i: (0, 0)),
        ],
        out_specs=[
            pl.BlockSpec((r2, in_features), lambda i: (i, 0)),
            pl.BlockSpec((r2, n_features), lambda i: (i, 0)),
        ],
        out_shape=[
            jax.ShapeDtypeStruct((n_features, in_features), jnp.float32),
            jax.ShapeDtypeStruct((out_features, n_features), jnp.float32),
        ],
    )(in_weights, out_weights, row_mask, mask)

    return (
        iw_new,
        ow_new,
        ru.reshape(n_features),
        racc_out.reshape(1),
        rage.reshape(n_features),
        (mask.reshape(n_features) != 0),
    )


# speedup vs baseline: 2.7669x; 1.2442x over previous
import functools

import numpy as np
import jax
import jax.numpy as jnp
from jax import lax
from jax.experimental import pallas as pl
from jax.experimental.pallas import tpu as pltpu

REPLACE_RATE = 1e-4
DECAY_RATE = 0.99
MATURITY_THRESHOLD = 100

_ROT = ((13, 15, 26, 6), (17, 29, 16, 24))


def _np_threefry2x32(k0, k1, x0, x1):
    x0 = x0.astype(np.uint32).copy()
    x1 = x1.astype(np.uint32).copy()
    ks = [np.uint32(k0), np.uint32(k1),
          np.uint32(np.uint32(k0) ^ np.uint32(k1) ^ np.uint32(0x1BD11BDA))]
    x0 = (x0 + ks[0]).astype(np.uint32)
    x1 = (x1 + ks[1]).astype(np.uint32)
    for i in range(5):
        for r in _ROT[i % 2]:
            x0 = (x0 + x1).astype(np.uint32)
            x1 = ((x1 << np.uint32(r)) | (x1 >> np.uint32(32 - r))).astype(np.uint32)
            x1 = (x1 ^ x0).astype(np.uint32)
        x0 = (x0 + ks[(i + 1) % 3]).astype(np.uint32)
        x1 = (x1 + ks[(i + 2) % 3] + np.uint32(i + 1)).astype(np.uint32)
    return x0, x1


def _in_key():
    b1, b2 = _np_threefry2x32(np.uint32(0), np.uint32(42),
                              np.array([0, 0], np.uint32), np.array([0, 1], np.uint32))
    return int(b1[0]), int(b2[0])


def _i32(v):
    return np.int32(np.uint32(v & 0xFFFFFFFF))


def _tf_bits(idx, k0, k1):
    ks = [_i32(k0), _i32(k1), _i32(k0 ^ k1 ^ 0x1BD11BDA)]
    x0 = jnp.full(idx.shape, ks[0], jnp.int32)
    x1 = idx + ks[1]
    for i in range(5):
        for r in _ROT[i % 2]:
            x0 = x0 + x1
            x1 = lax.shift_left(x1, np.int32(r)) | lax.shift_right_logical(x1, np.int32(32 - r))
            x1 = lax.bitwise_xor(x1, x0)
        x0 = x0 + ks[(i + 1) % 3]
        x1 = x1 + ks[(i + 2) % 3] + np.int32(i + 1)
    return lax.bitwise_xor(x0, x1)


def _uniform_from_bits(bits, limit):
    fb = lax.shift_right_logical(bits, np.int32(9)) | np.int32(0x3F800000)
    f = lax.bitcast_convert_type(fb, jnp.float32) - np.float32(1.0)
    return jnp.maximum(np.float32(-limit),
                       f * np.float32(2.0 * limit) + np.float32(-limit))


_POS_INF_BITS = np.int32(0x7F800000)


def _kth_smallest_bits(bits, k):
    def body(_, lohi):
        lo, hi = lohi
        mid = lo + lax.div(hi - lo, jnp.int32(2))
        cnt = jnp.sum((bits <= mid).astype(jnp.int32))
        ge = cnt >= k
        return (jnp.where(ge, lo, mid + 1), jnp.where(ge, mid, hi))
    _, hi = lax.fori_loop(0, 31, body, (jnp.int32(0), _POS_INF_BITS))
    return hi


def _stats_kernel(batch, n_features,
                  ow_ref, act_ref, util_ref, age_ref, racc_ref,
                  nu_ref, racc_out_ref, rage_ref, mask_ref, ow_out_ref,
                  ws_ref, im_ref):
    i = pl.program_id(0)
    ng = pl.num_programs(0)

    ow = ow_ref[...]
    ow_out_ref[...] = ow
    s = jnp.sum(jnp.abs(ow), axis=0, keepdims=True)
    a = jnp.sum(jnp.abs(act_ref[...]), axis=0, keepdims=True)

    @pl.when(i == 0)
    def _():
        ws_ref[...] = s
        im_ref[...] = a

    @pl.when(i > 0)
    def _():
        ws_ref[...] = ws_ref[...] + s
        im_ref[...] = im_ref[...] + a

    @pl.when(i == ng - 1)
    def _():
        im = im_ref[...] * np.float32(1.0 / batch)
        step_u = im * ws_ref[...]
        new_u = np.float32(1.0 - DECAY_RATE) * step_u + np.float32(DECAY_RATE) * util_ref[...]
        new_age = age_ref[...] + 1
        elig = new_age > MATURITY_THRESHOLD
        n_elig = jnp.sum(elig.astype(jnp.int32))
        racc1 = racc_ref[...][0, 0] + np.float32(REPLACE_RATE * n_features)
        n_av = racc1.astype(jnp.int32)
        k = jnp.minimum(n_av, n_elig)

        ubits = lax.bitcast_convert_type(new_u, jnp.int32)
        fbits = jnp.where(elig, ubits, _POS_INF_BITS)
        tbits = _kth_smallest_bits(fbits, k)

        pm = jnp.logical_and(jnp.logical_and(n_av > 0, elig), fbits <= tbits)
        nu_ref[...] = new_u
        rage_ref[...] = jnp.where(pm, 0, new_age)
        mask_ref[...] = pm.astype(jnp.int32)
        racc2 = racc1 - jnp.where(n_av > 0, k, 0).astype(jnp.float32)
        racc_out_ref[...] = jnp.full((1, 1), racc2, jnp.float32)


def _fixup_kernel(n_features, in_features, out_features, k0, k1, limit,
                  mask_ref, nu_ref, iw_in, ow_in, iw_ref, ow_ref, ru_ref,
                  row_scr, col_scr, row_sem, col_sem):
    pm = mask_ref[...] != 0
    pmi = pm.astype(jnp.int32)
    total = jnp.sum(pmi)
    iota = lax.broadcasted_iota(jnp.int32, (1, n_features), 1)
    masked_iota = jnp.where(pm, iota, _POS_INF_BITS)

    new_u = nu_ref[...]
    ubits = lax.bitcast_convert_type(new_u, jnp.int32)
    m_lo = _kth_smallest_bits(ubits, jnp.int32(n_features // 2))
    m_hi = _kth_smallest_bits(ubits, jnp.int32(n_features // 2 + 1))
    med = (lax.bitcast_convert_type(m_lo, jnp.float32)
           + lax.bitcast_convert_type(m_hi, jnp.float32)) * np.float32(0.5)
    ru_ref[...] = jnp.where(pm, med, new_u)

    def body(s, _):
        f = _kth_smallest_bits(masked_iota, s + 1)

        f0r = (f // 8) * 8
        rcp_in = pltpu.make_async_copy(iw_ref.at[pl.ds(f0r, 8), :], row_scr, row_sem)
        rcp_in.start()
        liota = lax.broadcasted_iota(jnp.int32, (8, in_features), 1)
        riota = lax.broadcasted_iota(jnp.int32, (8, in_features), 0)
        rng = _uniform_from_bits(_tf_bits(f * np.int32(in_features) + liota, k0, k1), limit)
        rcp_in.wait()
        row_scr[...] = jnp.where(riota == (f - f0r), rng, row_scr[...])
        rcp_out = pltpu.make_async_copy(row_scr, iw_ref.at[pl.ds(f0r, 8), :], row_sem)
        rcp_out.start()

        f0c = (f // 128) * 128
        ccp_in = pltpu.make_async_copy(ow_ref.at[:, pl.ds(f0c, 128)], col_scr, col_sem)
        ccp_in.start()
        ccp_in.wait()
        ciota = lax.broadcasted_iota(jnp.int32, (out_features, 128), 1)
        col_scr[...] = jnp.where(ciota == (f - f0c), np.float32(0.0), col_scr[...])
        ccp_out = pltpu.make_async_copy(col_scr, ow_ref.at[:, pl.ds(f0c, 128)], col_sem)
        ccp_out.start()
        rcp_out.wait()
        ccp_out.wait()
        return 0

    lax.fori_loop(0, total, body, 0)


def kernel(in_weights, out_weights, activation_values, utility, replacement_accumulator, age):
    n_features = out_weights.shape[1]
    out_features = out_weights.shape[0]
    in_features = in_weights.shape[1]
    batch = activation_values.shape[0]

    g1 = 8
    ow_rows = out_features // g1
    act_rows = batch // g1

    nu, racc_out, rage, mask, ow_pass = pl.pallas_call(
        functools.partial(_stats_kernel, batch, n_features),
        grid=(g1,),
        in_specs=[
            pl.BlockSpec((ow_rows, n_features), lambda i: (i, 0)),
            pl.BlockSpec((act_rows, n_features), lambda i: (i, 0)),
            pl.BlockSpec((1, n_features), lambda i: (0, 0)),
            pl.BlockSpec((1, n_features), lambda i: (0, 0)),
            pl.BlockSpec((1, 1), lambda i: (0, 0)),
        ],
        out_specs=[
            pl.BlockSpec((1, n_features), lambda i: (0, 0)),
            pl.BlockSpec((1, 1), lambda i: (0, 0)),
            pl.BlockSpec((1, n_features), lambda i: (0, 0)),
            pl.BlockSpec((1, n_features), lambda i: (0, 0)),
            pl.BlockSpec((ow_rows, n_features), lambda i: (i, 0)),
        ],
        out_shape=[
            jax.ShapeDtypeStruct((1, n_features), jnp.float32),
            jax.ShapeDtypeStruct((1, 1), jnp.float32),
            jax.ShapeDtypeStruct((1, n_features), jnp.int32),
            jax.ShapeDtypeStruct((1, n_features), jnp.int32),
            jax.ShapeDtypeStruct((out_features, n_features), jnp.float32),
        ],
        scratch_shapes=[
            pltpu.VMEM((1, n_features), jnp.float32),
            pltpu.VMEM((1, n_features), jnp.float32),
        ],
    )(
        out_weights,
        activation_values,
        utility.reshape(1, n_features),
        age.reshape(1, n_features),
        replacement_accumulator.reshape(1, 1),
    )

    k0, k1 = _in_key()
    limit = float(np.sqrt(np.float32(3.0) / np.float32(in_features)))

    iw_new, ow_new, ru = pl.pallas_call(
        functools.partial(_fixup_kernel, n_features, in_features, out_features, k0, k1, limit),
        in_specs=[
            pl.BlockSpec((1, n_features), lambda: (0, 0)),
            pl.BlockSpec((1, n_features), lambda: (0, 0)),
            pl.BlockSpec(memory_space=pl.ANY),
            pl.BlockSpec(memory_space=pl.ANY),
        ],
        out_specs=[
            pl.BlockSpec(memory_space=pl.ANY),
            pl.BlockSpec(memory_space=pl.ANY),
            pl.BlockSpec((1, n_features), lambda: (0, 0)),
        ],
        out_shape=[
            jax.ShapeDtypeStruct((n_features, in_features), jnp.float32),
            jax.ShapeDtypeStruct((out_features, n_features), jnp.float32),
            jax.ShapeDtypeStruct((1, n_features), jnp.float32),
        ],
        input_output_aliases={2: 0, 3: 1},
        scratch_shapes=[
            pltpu.VMEM((8, in_features), jnp.float32),
            pltpu.VMEM((out_features, 128), jnp.float32),
            pltpu.SemaphoreType.DMA,
            pltpu.SemaphoreType.DMA,
        ],
    )(mask, nu, in_weights, ow_pass)

    return (
        iw_new,
        ow_new,
        ru.reshape(n_features),
        racc_out.reshape(1),
        rage.reshape(n_features),
        (mask.reshape(n_features) != 0),
    )
